# baseline (speedup 1.0000x reference)
import functools
import jax
import jax.numpy as jnp
from jax import lax
from jax.experimental import pallas as pl
from jax.experimental.pallas import tpu as pltpu

B = 1024
D_MODEL = 1024
D_MEMORY = 64
NUM_HEADS = 8
BANK_SIZE = 16
MEMORY_SIZE = 1024
TOPK = 2
HD = NUM_HEADS * D_MEMORY


def _make_heads_kernel(sel_idx_ref, sel_probs_ref,
                       q_ref, s_ref, Wq_ref, bq_ref, Ws_ref, bs_ref,
                       qh_ref, sh_ref):
    n = pl.program_id(0)
    probs = sel_probs_ref[...]
    psum = jnp.sum(probs, axis=1, keepdims=True) + 1e-9
    pnorm = probs / psum
    sel = sel_idx_ref[...]
    cn = jnp.sum(jnp.where(sel == n, pnorm, 0.0), axis=1, keepdims=True)

    xq = q_ref[...]
    xs = s_ref[...]
    pq = jnp.dot(xq, Wq_ref[0], preferred_element_type=jnp.float32) + bq_ref[0]
    ps = jnp.dot(xs, Ws_ref[0], preferred_element_type=jnp.float32) + bs_ref[0]

    @pl.when(n == 0)
    def _():
        qh_ref[...] = cn * pq
        sh_ref[...] = cn * ps

    @pl.when(n > 0)
    def _():
        qh_ref[...] += cn * pq
        sh_ref[...] += cn * ps


def _memory_write_kernel(qh_ref, sh_ref, mem_ref, out_ref, *, nb):
    scale = 1.0 / (D_MEMORY ** 0.5)
    for i in range(nb):
        mem_i = mem_ref[i]
        q_i = qh_ref[i] * scale
        scores = lax.dot_general(mem_i, q_i,
                                 (((1,), (1,)), ((), ())),
                                 preferred_element_type=jnp.float32)
        m = jnp.max(scores, axis=0, keepdims=True)
        e = jnp.exp(scores - m)
        attn = e / jnp.sum(e, axis=0, keepdims=True)
        upd = lax.dot_general(attn, sh_ref[i],
                              (((1,), (0,)), ((), ())),
                              preferred_element_type=jnp.float32)
        out_ref[i] = mem_i + upd


def kernel(query, statement, memories, sel_probs, Wq, bq, Ws, bs, sel_indices):
    sel_indices = sel_indices.astype(jnp.int32)
    bq = bq.reshape(BANK_SIZE, 1, HD)
    bs = bs.reshape(BANK_SIZE, 1, HD)

    qh, sh = pl.pallas_call(
        _make_heads_kernel,
        grid=(BANK_SIZE,),
        in_specs=[
            pl.BlockSpec((B, TOPK), lambda n: (0, 0)),
            pl.BlockSpec((B, TOPK), lambda n: (0, 0)),
            pl.BlockSpec((B, D_MODEL), lambda n: (0, 0)),
            pl.BlockSpec((B, D_MODEL), lambda n: (0, 0)),
            pl.BlockSpec((1, D_MODEL, HD), lambda n: (n, 0, 0)),
            pl.BlockSpec((1, 1, HD), lambda n: (n, 0, 0)),
            pl.BlockSpec((1, D_MODEL, HD), lambda n: (n, 0, 0)),
            pl.BlockSpec((1, 1, HD), lambda n: (n, 0, 0)),
        ],
        out_specs=[
            pl.BlockSpec((B, HD), lambda n: (0, 0)),
            pl.BlockSpec((B, HD), lambda n: (0, 0)),
        ],
        out_shape=[
            jax.ShapeDtypeStruct((B, HD), jnp.float32),
            jax.ShapeDtypeStruct((B, HD), jnp.float32),
        ],
        compiler_params=pltpu.CompilerParams(
            dimension_semantics=("arbitrary",),
        ),
    )(sel_indices, sel_probs, query, statement, Wq, bq, Ws, bs)

    qh3 = qh.reshape(B, NUM_HEADS, D_MEMORY)
    sh3 = sh.reshape(B, NUM_HEADS, D_MEMORY)

    NB = 8
    out = pl.pallas_call(
        functools.partial(_memory_write_kernel, nb=NB),
        grid=(B // NB,),
        in_specs=[
            pl.BlockSpec((NB, NUM_HEADS, D_MEMORY), lambda i: (i, 0, 0)),
            pl.BlockSpec((NB, NUM_HEADS, D_MEMORY), lambda i: (i, 0, 0)),
            pl.BlockSpec((NB, MEMORY_SIZE, D_MEMORY), lambda i: (i, 0, 0)),
        ],
        out_specs=pl.BlockSpec((NB, MEMORY_SIZE, D_MEMORY), lambda i: (i, 0, 0)),
        out_shape=jax.ShapeDtypeStruct((B, MEMORY_SIZE, D_MEMORY), jnp.float32),
        compiler_params=pltpu.CompilerParams(
            dimension_semantics=("arbitrary",),
        ),
    )(qh3, sh3, memories)

    return out
